# baseline (speedup 1.0000x reference)
import functools

import jax
import jax.numpy as jnp
from jax import lax
from jax.experimental import pallas as pl
from jax.experimental.pallas import tpu as pltpu
from jax.experimental.pallas import tpu_sc as plsc

NC = 2
NS = 16
NW = NC * NS
L = 16

FB = 16
WPAD = 2 * L
BLK = 128


@functools.lru_cache(maxsize=None)
def _make_kernels(N, D, S):
    assert D % L == 0 and N % 8 == 0
    ND = D // L
    UNITS = N // 8
    CB = (UNITS // NW) * 8
    EXTRA = UNITS % NW
    CMAX = CB + (8 if EXTRA else 0)
    NFULL = CB // BLK
    TUNITS = (CMAX - NFULL * BLK) // 8
    GRP = BLK // L
    assert NFULL >= 2 and NFULL % 2 == 0 and BLK % L == 0
    assert CMAX - NFULL * BLK < BLK
    SLEN = ((CMAX + 2 * L + 15) // 16) * 16
    NPAD = (NW - 1) * CB + 8 * min(NW - 1, EXTRA) + SLEN
    TRASH = S
    SROWS = S + FB
    STRIPE = S // NS

    mesh = plsc.VectorSubcoreMesh(
        core_axis_name="c", subcore_axis_name="s", num_cores=NC, num_subcores=NS
    )

    def k1_body(seg_hbm, pos_hbm, h_hbm, w_hbm, num_out, den_out,
                num_acc, den_acc, h_buf, seg_v, pos_v, wrow_v, w_v,
                fnum, fden, zbuf, zbufd, acc_ref, dacc, fidx, scal,
                sem0, sem1):
        cid = lax.axis_index("c")
        sid = lax.axis_index("s")
        wid = cid * NS + sid

        zero = jnp.zeros((L,), jnp.float32)
        lane = lax.iota(jnp.int32, L)
        trash_vec = jnp.full((L,), TRASH, jnp.int32)
        one_i = jnp.full((L,), 1, jnp.int32)
        onehot0 = (one_i - jnp.minimum(lane, one_i)).astype(jnp.float32)

        start = pl.multiple_of(wid * CB + 8 * jnp.minimum(wid, EXTRA), 8)
        cnt = CB + 8 * (wid < EXTRA).astype(jnp.int32)

        sems = (sem0, sem1)

        def issue(b, par):
            pltpu.async_copy(
                h_hbm.at[pl.ds(pl.multiple_of((start + b * BLK) * D, 8),
                               BLK * D)],
                h_buf.at[par], sems[par])

        def wait(b, par):
            pltpu.make_async_copy(
                h_hbm.at[pl.ds(pl.multiple_of((start + b * BLK) * D, 8),
                               BLK * D)],
                h_buf.at[par], sems[par]).wait()

        issue(0, 0)

        pltpu.sync_copy(seg_hbm.at[pl.ds(start, SLEN)], seg_v)
        pltpu.sync_copy(pos_hbm.at[pl.ds(start, SLEN)], pos_v)
        pltpu.sync_copy(w_hbm, w_v)
        wv0 = w_v[pl.ds(0, L)]
        w0, w1, w2 = wv0[0], wv0[1], wv0[2]

        def pack(g, _):
            pv = pos_v[pl.ds(g * L, L)]
            a1 = jnp.minimum(pv, one_i).astype(jnp.float32)
            a2 = jnp.maximum(pv - one_i, 0).astype(jnp.float32)
            wrow_v[pl.ds(g * L, L)] = w0 + a1 * (w1 - w0) + a2 * (w2 - w1)
            return 0
        lax.fori_loop(0, SLEN // L, pack, 0)

        def zrow(r, _):
            zr, zdr = zbuf.at[r], zbufd.at[r]
            for k in range(ND):
                zr[pl.ds(k * L, L)] = zero
                zdr[pl.ds(k * L, L)] = zero
            return 0
        lax.fori_loop(0, STRIPE, zrow, 0)
        pltpu.sync_copy(zbuf, num_acc.at[pl.ds(sid * STRIPE, STRIPE)])
        pltpu.sync_copy(zbufd, den_acc.at[pl.ds(sid * STRIPE, STRIPE)])
        plsc.subcore_barrier()

        def flush_and_reset(cur, cnt16):
            fr = fnum.at[cnt16]
            for k in range(ND):
                fr[pl.ds(k * L, L)] = acc_ref[pl.ds(k * L, L)]
                acc_ref[pl.ds(k * L, L)] = zero
            densv = dacc[pl.ds(0, L)]
            dacc[pl.ds(0, L)] = zero
            dtot = densv[0]
            for j in range(1, L):
                dtot = dtot + densv[j]
            fd = fden.at[cnt16]
            fd[pl.ds(0, L)] = dtot * onehot0
            mm = one_i - jnp.minimum(jnp.abs(lane - cnt16), one_i)
            fv = fidx[pl.ds(0, L)]
            fidx[pl.ds(0, L)] = fv * (one_i - mm) + cur * mm
            scal[1] = jnp.where(cnt16 == FB - 1, 0, cnt16 + 1)

            @pl.when(cnt16 == FB - 1)
            def _():
                send_batch()

        def send_batch():
            pltpu.sync_copy(fnum, num_acc.at[fidx], add=True)
            pltpu.sync_copy(fden, den_acc.at[fidx], add=True)
            fidx[pl.ds(0, L)] = trash_vec

        def make_row(hb, off, cutoff=None):
            def row(i, _):
                cur = scal[0]
                s = seg_v[pl.ds(i, L)][0]
                w = wrow_v[pl.ds(i, L)][0]
                if cutoff is not None:
                    valid = i < cutoff
                    s = jnp.where(valid, s, cur)
                    w = jnp.where(valid, w, 0.0)
                hbase = (i - off) * D

                @pl.when(s != cur)
                def _():
                    flush_and_reset(cur, scal[1])
                    scal[0] = s

                for k in range(ND):
                    acc_ref[pl.ds(k * L, L)] = (
                        acc_ref[pl.ds(k * L, L)]
                        + w * hb[pl.ds(hbase + k * L, L)])
                dacc[pl.ds(0, L)] = dacc[pl.ds(0, L)] + w * onehot0
                return 0
            return row

        def make_block(par):
            hb = h_buf.at[par]

            def group(g, b):
                cur = scal[0]
                grow = b * BLK + g * L
                sv = seg_v[pl.ds(grow, L)]
                wvec = wrow_v[pl.ds(grow, L)]
                s_first = sv[0]
                anyb = (s_first != cur) | (s_first != sv[L - 1])

                @pl.when(jnp.logical_not(anyb))
                def _():
                    accs = [acc_ref[pl.ds(k * L, L)] for k in range(ND)]
                    for r in range(L):
                        wr = wvec[r]
                        hbase = (g * L + r) * D
                        accs = [accs[k] + wr * hb[pl.ds(hbase + k * L, L)]
                                for k in range(ND)]
                    for k in range(ND):
                        acc_ref[pl.ds(k * L, L)] = accs[k]
                    dacc[pl.ds(0, L)] = dacc[pl.ds(0, L)] + wvec

                @pl.when(anyb)
                def _():
                    lax.fori_loop(grow, grow + L, make_row(hb, b * BLK), 0)
                return b

            def block(b):
                lax.fori_loop(0, GRP, group, b, unroll=4)
            return block

        for k in range(ND):
            acc_ref[pl.ds(k * L, L)] = zero
        dacc[pl.ds(0, L)] = zero
        fidx[pl.ds(0, L)] = trash_vec
        scal[0] = seg_v[pl.ds(0, L)][0]
        scal[1] = 0

        block0 = make_block(0)
        block1 = make_block(1)

        def two_blocks(j, _):
            b0 = j * 2
            wait(b0, 0)
            issue(b0 + 1, 1)
            block0(b0)
            wait(b0 + 1, 1)

            @pl.when(b0 + 2 < NFULL)
            def _():
                issue(b0 + 2, 0)
            block1(b0 + 1)
            return 0

        lax.fori_loop(0, NFULL // 2, two_blocks, 0)

        tb = h_buf.at[0]
        for u in range(TUNITS):
            trow = NFULL * BLK + u * 8

            @pl.when(trow < cnt)
            def _(u=u, trow=trow):
                pltpu.sync_copy(
                    h_hbm.at[pl.ds(pl.multiple_of((start + trow) * D, 8),
                                   8 * D)],
                    tb.at[pl.ds(u * 8 * D, 8 * D)])
        lax.fori_loop(NFULL * BLK, CMAX,
                      make_row(tb, NFULL * BLK, cnt), 0)

        flush_and_reset(scal[0], scal[1])
        send_batch()

        plsc.subcore_barrier()
        pltpu.sync_copy(num_acc.at[pl.ds(sid * STRIPE, STRIPE)],
                        num_out.at[cid, pl.ds(sid * STRIPE, STRIPE)])
        pltpu.sync_copy(den_acc.at[pl.ds(sid * STRIPE, STRIPE)],
                        den_out.at[cid, pl.ds(sid * STRIPE, STRIPE)])

    k1 = pl.kernel(
        k1_body,
        out_type=(
            jax.ShapeDtypeStruct((NC, S, D), jnp.float32),
            jax.ShapeDtypeStruct((NC, S, D), jnp.float32),
        ),
        mesh=mesh,
        scratch_types=[
            pltpu.VMEM_SHARED((SROWS, D), jnp.float32),
            pltpu.VMEM_SHARED((SROWS, D), jnp.float32),
            pltpu.VMEM((2, BLK * D), jnp.float32),
            pltpu.VMEM((SLEN,), jnp.int32),
            pltpu.VMEM((SLEN,), jnp.int32),
            pltpu.VMEM((SLEN,), jnp.float32),
            pltpu.VMEM((WPAD,), jnp.float32),
            pltpu.VMEM((FB, D), jnp.float32),
            pltpu.VMEM((FB, D), jnp.float32),
            pltpu.VMEM((STRIPE, D), jnp.float32),
            pltpu.VMEM((STRIPE, D), jnp.float32),
            pltpu.VMEM((D,), jnp.float32),
            pltpu.VMEM((L,), jnp.float32),
            pltpu.VMEM((L,), jnp.int32),
            pltpu.SMEM((8,), jnp.int32),
            pltpu.SemaphoreType.DMA,
            pltpu.SemaphoreType.DMA,
        ],
        name="wmean_segsum",
    )

    R2 = S // NW

    def k2_body(num_hbm, den_hbm, out_hbm, n0, n1, d0, d1, ov):
        cid = lax.axis_index("c")
        sid = lax.axis_index("s")
        wid = cid * NS + sid
        base = wid * R2
        pltpu.sync_copy(num_hbm.at[0, pl.ds(base, R2)], n0)
        pltpu.sync_copy(num_hbm.at[1, pl.ds(base, R2)], n1)
        pltpu.sync_copy(den_hbm.at[0, pl.ds(base, R2)], d0)
        pltpu.sync_copy(den_hbm.at[1, pl.ds(base, R2)], d1)

        def row(r, _):
            d = d0.at[r][pl.ds(0, L)] + d1.at[r][pl.ds(0, L)]
            inv = (1.0 / jnp.where(d > 0, d, 1.0))[0]
            n0r, n1r, ovr = n0.at[r], n1.at[r], ov.at[r]
            for k in range(ND):
                ovr[pl.ds(k * L, L)] = (
                    n0r[pl.ds(k * L, L)] + n1r[pl.ds(k * L, L)]
                ) * inv
            return 0
        lax.fori_loop(0, R2, row, 0)
        pltpu.sync_copy(ov, out_hbm.at[pl.ds(base, R2)])

    k2 = pl.kernel(
        k2_body,
        out_type=jax.ShapeDtypeStruct((S, D), jnp.float32),
        mesh=mesh,
        scratch_types=[
            pltpu.VMEM((R2, D), jnp.float32),
            pltpu.VMEM((R2, D), jnp.float32),
            pltpu.VMEM((R2, D), jnp.float32),
            pltpu.VMEM((R2, D), jnp.float32),
            pltpu.VMEM((R2, D), jnp.float32),
        ],
        name="wmean_combine",
    )

    return k1, k2, NPAD


def kernel(h, pos, segment_ids, pos_table):
    N, D = h.shape
    S = 1024
    k1, k2, npad = _make_kernels(N, D, S)

    seg32 = segment_ids.astype(jnp.int32)
    pos32 = pos.astype(jnp.int32)
    w = jax.nn.softplus(pos_table.astype(jnp.float32)).reshape(-1)
    wpad = jnp.pad(w, (0, WPAD - w.shape[0]))
    seg_p = jnp.pad(seg32, (0, npad - N))
    pos_p = jnp.pad(pos32, (0, npad - N))

    num, den = k1(seg_p, pos_p, h.reshape(-1), wpad)
    return k2(num, den)


# speedup vs baseline: 1.1632x; 1.1632x over previous
import functools

import jax
import jax.numpy as jnp
from jax import lax
from jax.experimental import pallas as pl
from jax.experimental.pallas import tpu as pltpu
from jax.experimental.pallas import tpu_sc as plsc

NC = 2
NS = 16
NW = NC * NS
L = 16

FB = 16
WPAD = 2 * L
BLK = 128


@functools.lru_cache(maxsize=None)
def _make_kernels(N, D, S):
    assert D % L == 0 and N % 8 == 0
    ND = D // L
    UNITS = N // 8
    CB = (UNITS // NW) * 8
    EXTRA = UNITS % NW
    CMAX = CB + (8 if EXTRA else 0)
    NFULL = CB // BLK
    TUNITS = (CMAX - NFULL * BLK) // 8
    GRP = BLK // L
    assert NFULL >= 2 and NFULL % 2 == 0 and BLK % L == 0
    assert CMAX - NFULL * BLK < BLK
    SLEN = ((CMAX + 2 * L + 15) // 16) * 16
    NPAD = (NW - 1) * CB + 8 * min(NW - 1, EXTRA) + SLEN
    TRASH = S
    SROWS = S + FB
    STRIPE = S // NS

    mesh = plsc.VectorSubcoreMesh(
        core_axis_name="c", subcore_axis_name="s", num_cores=NC, num_subcores=NS
    )

    def k1_body(seg_hbm, pos_hbm, h_hbm, w_hbm, num_out, den_out,
                num_acc, den_acc, h_buf, seg_v, pos_v, wrow_v, w_v,
                fnum, fden, zbuf, zbufd, acc_ref, dacc, fidx, scal,
                sem0, sem1):
        cid = lax.axis_index("c")
        sid = lax.axis_index("s")
        wid = cid * NS + sid

        zero = jnp.zeros((L,), jnp.float32)
        lane = lax.iota(jnp.int32, L)
        trash_vec = jnp.full((L,), TRASH, jnp.int32)
        one_i = jnp.full((L,), 1, jnp.int32)
        onehot0 = (one_i - jnp.minimum(lane, one_i)).astype(jnp.float32)

        start = pl.multiple_of(wid * CB + 8 * jnp.minimum(wid, EXTRA), 8)
        cnt = CB + 8 * (wid < EXTRA).astype(jnp.int32)

        sems = (sem0, sem1)

        def issue(b, par):
            pltpu.async_copy(
                h_hbm.at[pl.ds(pl.multiple_of((start + b * BLK) * D, 8),
                               BLK * D)],
                h_buf.at[par], sems[par])

        def wait(b, par):
            pltpu.make_async_copy(
                h_hbm.at[pl.ds(pl.multiple_of((start + b * BLK) * D, 8),
                               BLK * D)],
                h_buf.at[par], sems[par]).wait()

        issue(0, 0)

        pltpu.sync_copy(seg_hbm.at[pl.ds(start, SLEN)], seg_v)
        pltpu.sync_copy(pos_hbm.at[pl.ds(start, SLEN)], pos_v)
        pltpu.sync_copy(w_hbm, w_v)
        wv0 = w_v[pl.ds(0, L)]
        w0, w1, w2 = wv0[0], wv0[1], wv0[2]

        def pack(g, _):
            pv = pos_v[pl.ds(g * L, L)]
            a1 = jnp.minimum(pv, one_i).astype(jnp.float32)
            a2 = jnp.maximum(pv - one_i, 0).astype(jnp.float32)
            wrow_v[pl.ds(g * L, L)] = w0 + a1 * (w1 - w0) + a2 * (w2 - w1)
            return 0
        lax.fori_loop(0, SLEN // L, pack, 0)

        def zrow(r, _):
            zr, zdr = zbuf.at[r], zbufd.at[r]
            for k in range(ND):
                zr[pl.ds(k * L, L)] = zero
                zdr[pl.ds(k * L, L)] = zero
            return 0
        lax.fori_loop(0, STRIPE, zrow, 0)
        pltpu.sync_copy(zbuf, num_acc.at[pl.ds(sid * STRIPE, STRIPE)])
        pltpu.sync_copy(zbufd, den_acc.at[pl.ds(sid * STRIPE, STRIPE)])
        plsc.subcore_barrier()

        def flush_and_reset(cur, cnt16):
            fr = fnum.at[cnt16]
            for k in range(ND):
                fr[pl.ds(k * L, L)] = acc_ref[pl.ds(k * L, L)]
                acc_ref[pl.ds(k * L, L)] = zero
            densv = dacc[pl.ds(0, L)]
            dacc[pl.ds(0, L)] = zero
            dtot = densv[0]
            for j in range(1, L):
                dtot = dtot + densv[j]
            fd = fden.at[cnt16]
            fd[pl.ds(0, L)] = dtot * onehot0
            mm = one_i - jnp.minimum(jnp.abs(lane - cnt16), one_i)
            fv = fidx[pl.ds(0, L)]
            fidx[pl.ds(0, L)] = fv * (one_i - mm) + cur * mm
            scal[1] = jnp.where(cnt16 == FB - 1, 0, cnt16 + 1)

            @pl.when(cnt16 == FB - 1)
            def _():
                send_batch()

        def send_batch():
            pltpu.sync_copy(fnum, num_acc.at[fidx], add=True)
            pltpu.sync_copy(fden, den_acc.at[fidx], add=True)
            fidx[pl.ds(0, L)] = trash_vec

        def make_row(hb, off, cutoff=None):
            def row(i, _):
                cur = scal[0]
                s = seg_v[pl.ds(i, L)][0]
                w = wrow_v[pl.ds(i, L)][0]
                if cutoff is not None:
                    valid = i < cutoff
                    s = jnp.where(valid, s, cur)
                    w = jnp.where(valid, w, 0.0)
                hbase = (i - off) * D

                @pl.when(s != cur)
                def _():
                    flush_and_reset(cur, scal[1])
                    scal[0] = s

                for k in range(ND):
                    acc_ref[pl.ds(k * L, L)] = (
                        acc_ref[pl.ds(k * L, L)]
                        + w * hb[pl.ds(hbase + k * L, L)])
                dacc[pl.ds(0, L)] = dacc[pl.ds(0, L)] + w * onehot0
                return 0
            return row

        def make_block(par):
            hb = h_buf.at[par]

            def group(g, b):
                cur = scal[0]
                grow = b * BLK + g * L
                sv = seg_v[pl.ds(grow, L)]
                wvec = wrow_v[pl.ds(grow, L)]
                s_first = sv[0]
                anyb = (s_first != cur) | (s_first != sv[L - 1])

                @pl.when(jnp.logical_not(anyb))
                def _():
                    accs = [acc_ref[pl.ds(k * L, L)] for k in range(ND)]
                    for r in range(L):
                        wr = wvec[r]
                        hbase = (g * L + r) * D
                        accs = [accs[k] + wr * hb[pl.ds(hbase + k * L, L)]
                                for k in range(ND)]
                    for k in range(ND):
                        acc_ref[pl.ds(k * L, L)] = accs[k]
                    dacc[pl.ds(0, L)] = dacc[pl.ds(0, L)] + wvec

                @pl.when(anyb)
                def _():
                    lax.fori_loop(grow, grow + L, make_row(hb, b * BLK), 0)
                return b

            def block(b):
                lax.fori_loop(0, GRP, group, b, unroll=2)
            return block

        for k in range(ND):
            acc_ref[pl.ds(k * L, L)] = zero
        dacc[pl.ds(0, L)] = zero
        fidx[pl.ds(0, L)] = trash_vec
        scal[0] = seg_v[pl.ds(0, L)][0]
        scal[1] = 0

        block0 = make_block(0)
        block1 = make_block(1)

        def two_blocks(j, _):
            b0 = j * 2
            wait(b0, 0)
            issue(b0 + 1, 1)
            block0(b0)
            wait(b0 + 1, 1)

            @pl.when(b0 + 2 < NFULL)
            def _():
                issue(b0 + 2, 0)
            block1(b0 + 1)
            return 0

        lax.fori_loop(0, NFULL // 2, two_blocks, 0)

        tb = h_buf.at[0]
        for u in range(TUNITS):
            trow = NFULL * BLK + u * 8

            @pl.when(trow < cnt)
            def _(u=u, trow=trow):
                pltpu.sync_copy(
                    h_hbm.at[pl.ds(pl.multiple_of((start + trow) * D, 8),
                                   8 * D)],
                    tb.at[pl.ds(u * 8 * D, 8 * D)])
        lax.fori_loop(NFULL * BLK, CMAX,
                      make_row(tb, NFULL * BLK, cnt), 0)

        flush_and_reset(scal[0], scal[1])
        send_batch()

        plsc.subcore_barrier()
        pltpu.sync_copy(num_acc.at[pl.ds(sid * STRIPE, STRIPE)],
                        num_out.at[cid, pl.ds(sid * STRIPE, STRIPE)])
        pltpu.sync_copy(den_acc.at[pl.ds(sid * STRIPE, STRIPE)],
                        den_out.at[cid, pl.ds(sid * STRIPE, STRIPE)])

    k1 = pl.kernel(
        k1_body,
        out_type=(
            jax.ShapeDtypeStruct((NC, S, D), jnp.float32),
            jax.ShapeDtypeStruct((NC, S, D), jnp.float32),
        ),
        mesh=mesh,
        scratch_types=[
            pltpu.VMEM_SHARED((SROWS, D), jnp.float32),
            pltpu.VMEM_SHARED((SROWS, D), jnp.float32),
            pltpu.VMEM((2, BLK * D), jnp.float32),
            pltpu.VMEM((SLEN,), jnp.int32),
            pltpu.VMEM((SLEN,), jnp.int32),
            pltpu.VMEM((SLEN,), jnp.float32),
            pltpu.VMEM((WPAD,), jnp.float32),
            pltpu.VMEM((FB, D), jnp.float32),
            pltpu.VMEM((FB, D), jnp.float32),
            pltpu.VMEM((STRIPE, D), jnp.float32),
            pltpu.VMEM((STRIPE, D), jnp.float32),
            pltpu.VMEM((D,), jnp.float32),
            pltpu.VMEM((L,), jnp.float32),
            pltpu.VMEM((L,), jnp.int32),
            pltpu.SMEM((8,), jnp.int32),
            pltpu.SemaphoreType.DMA,
            pltpu.SemaphoreType.DMA,
        ],
        name="wmean_segsum",
    )

    R2 = S // NW

    def k2_body(num_hbm, den_hbm, out_hbm, n0, n1, d0, d1, ov):
        cid = lax.axis_index("c")
        sid = lax.axis_index("s")
        wid = cid * NS + sid
        base = wid * R2
        pltpu.sync_copy(num_hbm.at[0, pl.ds(base, R2)], n0)
        pltpu.sync_copy(num_hbm.at[1, pl.ds(base, R2)], n1)
        pltpu.sync_copy(den_hbm.at[0, pl.ds(base, R2)], d0)
        pltpu.sync_copy(den_hbm.at[1, pl.ds(base, R2)], d1)

        def row(r, _):
            d = d0.at[r][pl.ds(0, L)] + d1.at[r][pl.ds(0, L)]
            inv = (1.0 / jnp.where(d > 0, d, 1.0))[0]
            n0r, n1r, ovr = n0.at[r], n1.at[r], ov.at[r]
            for k in range(ND):
                ovr[pl.ds(k * L, L)] = (
                    n0r[pl.ds(k * L, L)] + n1r[pl.ds(k * L, L)]
                ) * inv
            return 0
        lax.fori_loop(0, R2, row, 0)
        pltpu.sync_copy(ov, out_hbm.at[pl.ds(base, R2)])

    k2 = pl.kernel(
        k2_body,
        out_type=jax.ShapeDtypeStruct((S, D), jnp.float32),
        mesh=mesh,
        scratch_types=[
            pltpu.VMEM((R2, D), jnp.float32),
            pltpu.VMEM((R2, D), jnp.float32),
            pltpu.VMEM((R2, D), jnp.float32),
            pltpu.VMEM((R2, D), jnp.float32),
            pltpu.VMEM((R2, D), jnp.float32),
        ],
        name="wmean_combine",
    )

    return k1, k2, NPAD


def kernel(h, pos, segment_ids, pos_table):
    N, D = h.shape
    S = 1024
    k1, k2, npad = _make_kernels(N, D, S)

    seg32 = segment_ids.astype(jnp.int32)
    pos32 = pos.astype(jnp.int32)
    w = jax.nn.softplus(pos_table.astype(jnp.float32)).reshape(-1)
    wpad = jnp.pad(w, (0, WPAD - w.shape[0]))
    seg_p = jnp.pad(seg32, (0, npad - N))
    pos_p = jnp.pad(pos32, (0, npad - N))

    num, den = k1(seg_p, pos_p, h.reshape(-1), wpad)
    return k2(num, den)
